# Initial kernel scaffold; baseline (speedup 1.0000x reference)
import jax
import jax.numpy as jnp
from jax.experimental import pallas as pl


def kernel(x, fp_w, fp_b, fp_bn_g, fp_bn_b, gat_wl, gat_wr, gat_att, gat_bias, gin_w, gin_b, gin_bn_g, gin_bn_b, ln_g, ln_b, res_w, res_b, gate_w1, gate_b1, gate_w2, gate_b2, h_qkv_w, h_qkv_b, h_out_w, h_out_b, h_w1, h_b1, h_bn_g, h_bn_b, h_w2, h_b2, edge_index, batch):
    raise NotImplementedError("write your pallas kernel here")


# speedup vs baseline: 1.0788x; 1.0788x over previous
import functools

import jax
import jax.numpy as jnp
import numpy as np
from jax.experimental import pallas as pl
from jax.experimental.pallas import tpu as pltpu

N = 10000
E = 160000
G = 64
D_IN = 256
D_H = 512
HEADS = 8
D_HEAD = D_H // HEADS
N_LABELS = 16
MHA_HEADS = 4
MHA_DH = D_H // MHA_HEADS

NB = 1000
N_BLOCKS = N // NB


def _leaky(x, s=0.2):
    return jnp.where(x >= 0, x, s * x)


def _featurize_body(x_ref, fp_w_ref, fp_b_ref, fp_bn_g_ref, fp_bn_b_ref,
                    gat_wl_ref, gat_wr_ref, xl_ref, xr_ref):
    h = jnp.maximum(jnp.dot(x_ref[...], fp_w_ref[...],
                            preferred_element_type=jnp.float32) + fp_b_ref[...], 0.0)
    h = fp_bn_g_ref[...] * h * (1.0 / np.sqrt(1.0 + 1e-5)) + fp_bn_b_ref[...]
    xl_ref[...] = jnp.dot(h, gat_wl_ref[...], preferred_element_type=jnp.float32)
    xr_ref[...] = jnp.dot(h, gat_wr_ref[...], preferred_element_type=jnp.float32)


def _featurize(x, fp_w, fp_b, fp_bn_g, fp_bn_b, gat_wl, gat_wr):
    return pl.pallas_call(
        _featurize_body,
        grid=(N_BLOCKS,),
        in_specs=[
            pl.BlockSpec((NB, D_IN), lambda i: (i, 0)),
            pl.BlockSpec((D_IN, D_H), lambda i: (0, 0)),
            pl.BlockSpec((D_H,), lambda i: (0,)),
            pl.BlockSpec((D_H,), lambda i: (0,)),
            pl.BlockSpec((D_H,), lambda i: (0,)),
            pl.BlockSpec((D_H, D_H), lambda i: (0, 0)),
            pl.BlockSpec((D_H, D_H), lambda i: (0, 0)),
        ],
        out_specs=[
            pl.BlockSpec((NB, D_H), lambda i: (i, 0)),
            pl.BlockSpec((NB, D_H), lambda i: (i, 0)),
        ],
        out_shape=[
            jax.ShapeDtypeStruct((N, D_H), jnp.float32),
            jax.ShapeDtypeStruct((N, D_H), jnp.float32),
        ],
    )(x, fp_w, fp_b, fp_bn_g, fp_bn_b, gat_wl, gat_wr)


def _node_mlp_body(h1_ref, agg_ref, batch_ref,
                   gin_w_ref, gin_b_ref, gin_bn_g_ref, gin_bn_b_ref,
                   ln_g_ref, ln_b_ref, res_w_ref, res_b_ref,
                   gate_w1_ref, gate_b1_ref, gate_w2_ref, gate_b2_ref,
                   embnum_ref, gs_ref):
    i = pl.program_id(0)
    z = h1_ref[...] + agg_ref[...]
    z = jnp.maximum(jnp.dot(z, gin_w_ref[...],
                            preferred_element_type=jnp.float32) + gin_b_ref[...], 0.0)
    z = gin_bn_g_ref[...] * z * (1.0 / np.sqrt(1.0 + 1e-5)) + gin_bn_b_ref[...]
    h = _leaky(z)
    x_res = jnp.dot(h, res_w_ref[...], preferred_element_type=jnp.float32) + res_b_ref[...]
    mu = jnp.mean(h, axis=-1, keepdims=True)
    var = jnp.mean((h - mu) ** 2, axis=-1, keepdims=True)
    hn = (h - mu) * jax.lax.rsqrt(var + 1e-5) * ln_g_ref[...] + ln_b_ref[...]
    h = _leaky(hn + x_res)
    gate = jnp.dot(jnp.tanh(jnp.dot(h, gate_w1_ref[...],
                                    preferred_element_type=jnp.float32) + gate_b1_ref[...]),
                   gate_w2_ref[...], preferred_element_type=jnp.float32) + gate_b2_ref[...]
    ge = jnp.exp(gate)
    b = batch_ref[0, 0, :]
    onehot = (b[None, :] == jax.lax.broadcasted_iota(jnp.int32, (G, NB), 0)
              ).astype(jnp.float32)
    geh = ge * h
    part_emb = jnp.dot(onehot, geh, preferred_element_type=jnp.float32)
    part_gs = jnp.dot(onehot, ge, preferred_element_type=jnp.float32)

    @pl.when(i == 0)
    def _():
        embnum_ref[...] = jnp.zeros_like(embnum_ref)
        gs_ref[...] = jnp.zeros_like(gs_ref)

    embnum_ref[...] += part_emb
    gs_ref[...] += part_gs


def _node_mlp_pool(h1, agg, batch3, gin_w, gin_b, gin_bn_g, gin_bn_b,
                   ln_g, ln_b, res_w, res_b, gate_w1, gate_b1, gate_w2, gate_b2):
    return pl.pallas_call(
        _node_mlp_body,
        grid=(N_BLOCKS,),
        in_specs=[
            pl.BlockSpec((NB, D_H), lambda i: (i, 0)),
            pl.BlockSpec((NB, D_H), lambda i: (i, 0)),
            pl.BlockSpec((1, 1, NB), lambda i: (i, 0, 0)),
            pl.BlockSpec((D_H, D_H), lambda i: (0, 0)),
            pl.BlockSpec((D_H,), lambda i: (0,)),
            pl.BlockSpec((D_H,), lambda i: (0,)),
            pl.BlockSpec((D_H,), lambda i: (0,)),
            pl.BlockSpec((D_H,), lambda i: (0,)),
            pl.BlockSpec((D_H,), lambda i: (0,)),
            pl.BlockSpec((D_H, D_H), lambda i: (0, 0)),
            pl.BlockSpec((D_H,), lambda i: (0,)),
            pl.BlockSpec((D_H, D_H), lambda i: (0, 0)),
            pl.BlockSpec((D_H,), lambda i: (0,)),
            pl.BlockSpec((D_H, 1), lambda i: (0, 0)),
            pl.BlockSpec((1,), lambda i: (0,)),
        ],
        out_specs=[
            pl.BlockSpec((G, D_H), lambda i: (0, 0)),
            pl.BlockSpec((G, 1), lambda i: (0, 0)),
        ],
        out_shape=[
            jax.ShapeDtypeStruct((G, D_H), jnp.float32),
            jax.ShapeDtypeStruct((G, 1), jnp.float32),
        ],
    )(h1, agg, batch3, gin_w, gin_b, gin_bn_g, gin_bn_b,
      ln_g, ln_b, res_w, res_b, gate_w1, gate_b1, gate_w2, gate_b2)


def _heads_body(embnum_ref, gs_ref, qkv_w_ref, qkv_b_ref, out_w_ref, out_b_ref,
                w1_ref, b1_ref, bn_g_ref, bn_b_ref, w2_ref, b2_ref, out_ref):
    emb = embnum_ref[...] / (gs_ref[...] + 1e-16)
    qkv = jnp.dot(emb, qkv_w_ref[0], preferred_element_type=jnp.float32) + qkv_b_ref[0]
    scale = 1.0 / np.sqrt(MHA_DH)
    os = []
    for hh in range(MHA_HEADS):
        q = qkv[:, hh * MHA_DH:(hh + 1) * MHA_DH]
        k = qkv[:, D_H + hh * MHA_DH:D_H + (hh + 1) * MHA_DH]
        v = qkv[:, 2 * D_H + hh * MHA_DH:2 * D_H + (hh + 1) * MHA_DH]
        s = jax.lax.dot_general(q, k, (((1,), (1,)), ((), ())),
                                preferred_element_type=jnp.float32) * scale
        s = s - jnp.max(s, axis=-1, keepdims=True)
        p = jnp.exp(s)
        p = p / jnp.sum(p, axis=-1, keepdims=True)
        os.append(jnp.dot(p, v, preferred_element_type=jnp.float32))
    o = jnp.concatenate(os, axis=-1)
    o = jnp.dot(o, out_w_ref[0], preferred_element_type=jnp.float32) + out_b_ref[0]
    z2 = jnp.dot(o, w1_ref[0], preferred_element_type=jnp.float32) + b1_ref[0]
    z2 = z2 * jax.nn.sigmoid(z2)
    z2 = bn_g_ref[0] * z2 * (1.0 / np.sqrt(1.0 + 1e-5)) + bn_b_ref[0]
    out_ref[0, 0] = (jnp.dot(z2, w2_ref[0], preferred_element_type=jnp.float32)
                     + b2_ref[0])[:, 0]


def _label_heads(embnum, gs, h_qkv_w, h_qkv_b, h_out_w, h_out_b,
                 h_w1, h_b1, h_bn_g, h_bn_b, h_w2, h_b2):
    out = pl.pallas_call(
        _heads_body,
        grid=(N_LABELS,),
        in_specs=[
            pl.BlockSpec((G, D_H), lambda i: (0, 0)),
            pl.BlockSpec((G, 1), lambda i: (0, 0)),
            pl.BlockSpec((1, D_H, 3 * D_H), lambda i: (i, 0, 0)),
            pl.BlockSpec((1, 1, 3 * D_H), lambda i: (i, 0, 0)),
            pl.BlockSpec((1, D_H, D_H), lambda i: (i, 0, 0)),
            pl.BlockSpec((1, 1, D_H), lambda i: (i, 0, 0)),
            pl.BlockSpec((1, D_H, 256), lambda i: (i, 0, 0)),
            pl.BlockSpec((1, 1, 256), lambda i: (i, 0, 0)),
            pl.BlockSpec((1, 1, 256), lambda i: (i, 0, 0)),
            pl.BlockSpec((1, 1, 256), lambda i: (i, 0, 0)),
            pl.BlockSpec((1, 256, 1), lambda i: (i, 0, 0)),
            pl.BlockSpec((1, 1, 1), lambda i: (i, 0, 0)),
        ],
        out_specs=pl.BlockSpec((1, 1, G), lambda i: (i, 0, 0)),
        out_shape=jax.ShapeDtypeStruct((N_LABELS, 1, G), jnp.float32),
    )(embnum, gs, h_qkv_w, h_qkv_b[:, None, :], h_out_w, h_out_b[:, None, :],
      h_w1, h_b1[:, None, :], h_bn_g[:, None, :], h_bn_b[:, None, :],
      h_w2, h_b2[:, None, :])
    return out[:, 0].T[:, :, None]


def _edge_phase(xl, xr, gat_att, gat_bias, edge_index):
    src = edge_index[0]
    dst = edge_index[1]
    xl3 = xl.reshape(N, HEADS, D_HEAD)
    xr3 = xr.reshape(N, HEADS, D_HEAD)
    e = _leaky(xl3[src] + xr3[dst])
    logits = (e * gat_att[None, :, :]).sum(-1)
    a = jnp.exp(logits)
    denom = jax.ops.segment_sum(a, dst, num_segments=N)
    num = jax.ops.segment_sum(xl3[src] * a[:, :, None], dst, num_segments=N)
    h1 = (num / (denom[:, :, None] + 1e-16)).reshape(N, D_H) + gat_bias
    agg = jax.ops.segment_sum(h1[src], dst, num_segments=N)
    return h1, agg


def kernel(x, fp_w, fp_b, fp_bn_g, fp_bn_b, gat_wl, gat_wr, gat_att, gat_bias,
           gin_w, gin_b, gin_bn_g, gin_bn_b, ln_g, ln_b, res_w, res_b,
           gate_w1, gate_b1, gate_w2, gate_b2, h_qkv_w, h_qkv_b, h_out_w,
           h_out_b, h_w1, h_b1, h_bn_g, h_bn_b, h_w2, h_b2, edge_index, batch):
    xl, xr = _featurize(x, fp_w, fp_b, fp_bn_g, fp_bn_b, gat_wl, gat_wr)
    h1, agg = _edge_phase(xl, xr, gat_att, gat_bias, edge_index)
    batch3 = batch.reshape(N_BLOCKS, 1, NB)
    embnum, gs = _node_mlp_pool(h1, agg, batch3, gin_w, gin_b, gin_bn_g,
                                gin_bn_b, ln_g, ln_b, res_w, res_b,
                                gate_w1, gate_b1, gate_w2, gate_b2)
    return _label_heads(embnum, gs, h_qkv_w, h_qkv_b, h_out_w, h_out_b,
                        h_w1, h_b1, h_bn_g, h_bn_b, h_w2, h_b2)


# speedup vs baseline: 2.4412x; 2.2628x over previous
import functools

import jax
import jax.numpy as jnp
import numpy as np
from jax import lax
from jax.experimental import pallas as pl
from jax.experimental.pallas import tpu as pltpu
from jax.experimental.pallas import tpu_sc as plsc

N = 10000
E = 160000
G = 64
D_IN = 256
D_H = 512
HEADS = 8
D_HEAD = D_H // HEADS
N_LABELS = 16
MHA_HEADS = 4
MHA_DH = D_H // MHA_HEADS

NB = 1000
N_BLOCKS = N // NB


def _leaky(x, s=0.2):
    return jnp.where(x >= 0, x, s * x)


def _featurize_body(x_ref, fp_w_ref, fp_b_ref, fp_bn_g_ref, fp_bn_b_ref,
                    gat_wl_ref, gat_wr_ref, xl_ref, xr_ref):
    h = jnp.maximum(jnp.dot(x_ref[...], fp_w_ref[...],
                            preferred_element_type=jnp.float32) + fp_b_ref[...], 0.0)
    h = fp_bn_g_ref[...] * h * (1.0 / np.sqrt(1.0 + 1e-5)) + fp_bn_b_ref[...]
    xl_ref[...] = jnp.dot(h, gat_wl_ref[...], preferred_element_type=jnp.float32)
    xr_ref[...] = jnp.dot(h, gat_wr_ref[...], preferred_element_type=jnp.float32)


def _featurize(x, fp_w, fp_b, fp_bn_g, fp_bn_b, gat_wl, gat_wr):
    return pl.pallas_call(
        _featurize_body,
        grid=(N_BLOCKS,),
        in_specs=[
            pl.BlockSpec((NB, D_IN), lambda i: (i, 0)),
            pl.BlockSpec((D_IN, D_H), lambda i: (0, 0)),
            pl.BlockSpec((D_H,), lambda i: (0,)),
            pl.BlockSpec((D_H,), lambda i: (0,)),
            pl.BlockSpec((D_H,), lambda i: (0,)),
            pl.BlockSpec((D_H, D_H), lambda i: (0, 0)),
            pl.BlockSpec((D_H, D_H), lambda i: (0, 0)),
        ],
        out_specs=[
            pl.BlockSpec((NB, D_H), lambda i: (i, 0)),
            pl.BlockSpec((NB, D_H), lambda i: (i, 0)),
        ],
        out_shape=[
            jax.ShapeDtypeStruct((N, D_H), jnp.float32),
            jax.ShapeDtypeStruct((N, D_H), jnp.float32),
        ],
    )(x, fp_w, fp_b, fp_bn_g, fp_bn_b, gat_wl, gat_wr)


def _node_mlp_body(h1_ref, agg_ref, batch_ref,
                   gin_w_ref, gin_b_ref, gin_bn_g_ref, gin_bn_b_ref,
                   ln_g_ref, ln_b_ref, res_w_ref, res_b_ref,
                   gate_w1_ref, gate_b1_ref, gate_w2_ref, gate_b2_ref,
                   embnum_ref, gs_ref):
    i = pl.program_id(0)
    z = h1_ref[...] + agg_ref[...]
    z = jnp.maximum(jnp.dot(z, gin_w_ref[...],
                            preferred_element_type=jnp.float32) + gin_b_ref[...], 0.0)
    z = gin_bn_g_ref[...] * z * (1.0 / np.sqrt(1.0 + 1e-5)) + gin_bn_b_ref[...]
    h = _leaky(z)
    x_res = jnp.dot(h, res_w_ref[...], preferred_element_type=jnp.float32) + res_b_ref[...]
    mu = jnp.mean(h, axis=-1, keepdims=True)
    var = jnp.mean((h - mu) ** 2, axis=-1, keepdims=True)
    hn = (h - mu) * jax.lax.rsqrt(var + 1e-5) * ln_g_ref[...] + ln_b_ref[...]
    h = _leaky(hn + x_res)
    gate = jnp.dot(jnp.tanh(jnp.dot(h, gate_w1_ref[...],
                                    preferred_element_type=jnp.float32) + gate_b1_ref[...]),
                   gate_w2_ref[...], preferred_element_type=jnp.float32) + gate_b2_ref[...]
    ge = jnp.exp(gate)
    b = batch_ref[0, 0, :]
    onehot = (b[None, :] == jax.lax.broadcasted_iota(jnp.int32, (G, NB), 0)
              ).astype(jnp.float32)
    geh = ge * h
    part_emb = jnp.dot(onehot, geh, preferred_element_type=jnp.float32)
    part_gs = jnp.dot(onehot, ge, preferred_element_type=jnp.float32)

    @pl.when(i == 0)
    def _():
        embnum_ref[...] = jnp.zeros_like(embnum_ref)
        gs_ref[...] = jnp.zeros_like(gs_ref)

    embnum_ref[...] += part_emb
    gs_ref[...] += part_gs


def _node_mlp_pool(h1, agg, batch3, gin_w, gin_b, gin_bn_g, gin_bn_b,
                   ln_g, ln_b, res_w, res_b, gate_w1, gate_b1, gate_w2, gate_b2):
    return pl.pallas_call(
        _node_mlp_body,
        grid=(N_BLOCKS,),
        in_specs=[
            pl.BlockSpec((NB, D_H), lambda i: (i, 0)),
            pl.BlockSpec((NB, D_H), lambda i: (i, 0)),
            pl.BlockSpec((1, 1, NB), lambda i: (i, 0, 0)),
            pl.BlockSpec((D_H, D_H), lambda i: (0, 0)),
            pl.BlockSpec((D_H,), lambda i: (0,)),
            pl.BlockSpec((D_H,), lambda i: (0,)),
            pl.BlockSpec((D_H,), lambda i: (0,)),
            pl.BlockSpec((D_H,), lambda i: (0,)),
            pl.BlockSpec((D_H,), lambda i: (0,)),
            pl.BlockSpec((D_H, D_H), lambda i: (0, 0)),
            pl.BlockSpec((D_H,), lambda i: (0,)),
            pl.BlockSpec((D_H, D_H), lambda i: (0, 0)),
            pl.BlockSpec((D_H,), lambda i: (0,)),
            pl.BlockSpec((D_H, 1), lambda i: (0, 0)),
            pl.BlockSpec((1,), lambda i: (0,)),
        ],
        out_specs=[
            pl.BlockSpec((G, D_H), lambda i: (0, 0)),
            pl.BlockSpec((G, 1), lambda i: (0, 0)),
        ],
        out_shape=[
            jax.ShapeDtypeStruct((G, D_H), jnp.float32),
            jax.ShapeDtypeStruct((G, 1), jnp.float32),
        ],
    )(h1, agg, batch3, gin_w, gin_b, gin_bn_g, gin_bn_b,
      ln_g, ln_b, res_w, res_b, gate_w1, gate_b1, gate_w2, gate_b2)


def _heads_body(embnum_ref, gs_ref, qkv_w_ref, qkv_b_ref, out_w_ref, out_b_ref,
                w1_ref, b1_ref, bn_g_ref, bn_b_ref, w2_ref, b2_ref, out_ref):
    emb = embnum_ref[...] / (gs_ref[...] + 1e-16)
    qkv = jnp.dot(emb, qkv_w_ref[0], preferred_element_type=jnp.float32) + qkv_b_ref[0]
    scale = 1.0 / np.sqrt(MHA_DH)
    os = []
    for hh in range(MHA_HEADS):
        q = qkv[:, hh * MHA_DH:(hh + 1) * MHA_DH]
        k = qkv[:, D_H + hh * MHA_DH:D_H + (hh + 1) * MHA_DH]
        v = qkv[:, 2 * D_H + hh * MHA_DH:2 * D_H + (hh + 1) * MHA_DH]
        s = jax.lax.dot_general(q, k, (((1,), (1,)), ((), ())),
                                preferred_element_type=jnp.float32) * scale
        s = s - jnp.max(s, axis=-1, keepdims=True)
        p = jnp.exp(s)
        p = p / jnp.sum(p, axis=-1, keepdims=True)
        os.append(jnp.dot(p, v, preferred_element_type=jnp.float32))
    o = jnp.concatenate(os, axis=-1)
    o = jnp.dot(o, out_w_ref[0], preferred_element_type=jnp.float32) + out_b_ref[0]
    z2 = jnp.dot(o, w1_ref[0], preferred_element_type=jnp.float32) + b1_ref[0]
    z2 = z2 * jax.nn.sigmoid(z2)
    z2 = bn_g_ref[0] * z2 * (1.0 / np.sqrt(1.0 + 1e-5)) + bn_b_ref[0]
    out_ref[0, 0] = (jnp.dot(z2, w2_ref[0], preferred_element_type=jnp.float32)
                     + b2_ref[0])[:, 0]


def _label_heads(embnum, gs, h_qkv_w, h_qkv_b, h_out_w, h_out_b,
                 h_w1, h_b1, h_bn_g, h_bn_b, h_w2, h_b2):
    out = pl.pallas_call(
        _heads_body,
        grid=(N_LABELS,),
        in_specs=[
            pl.BlockSpec((G, D_H), lambda i: (0, 0)),
            pl.BlockSpec((G, 1), lambda i: (0, 0)),
            pl.BlockSpec((1, D_H, 3 * D_H), lambda i: (i, 0, 0)),
            pl.BlockSpec((1, 1, 3 * D_H), lambda i: (i, 0, 0)),
            pl.BlockSpec((1, D_H, D_H), lambda i: (i, 0, 0)),
            pl.BlockSpec((1, 1, D_H), lambda i: (i, 0, 0)),
            pl.BlockSpec((1, D_H, 256), lambda i: (i, 0, 0)),
            pl.BlockSpec((1, 1, 256), lambda i: (i, 0, 0)),
            pl.BlockSpec((1, 1, 256), lambda i: (i, 0, 0)),
            pl.BlockSpec((1, 1, 256), lambda i: (i, 0, 0)),
            pl.BlockSpec((1, 256, 1), lambda i: (i, 0, 0)),
            pl.BlockSpec((1, 1, 1), lambda i: (i, 0, 0)),
        ],
        out_specs=pl.BlockSpec((1, 1, G), lambda i: (i, 0, 0)),
        out_shape=jax.ShapeDtypeStruct((N_LABELS, 1, G), jnp.float32),
    )(embnum, gs, h_qkv_w, h_qkv_b[:, None, :], h_out_w, h_out_b[:, None, :],
      h_w1, h_b1[:, None, :], h_bn_g[:, None, :], h_bn_b[:, None, :],
      h_w2, h_b2[:, None, :])
    return out[:, 0].T[:, :, None]


NPAD = 10240
NW = 32
RNG = 128
NRANGE = NPAD // RNG
NROUND = 3
ACCR = RNG + 1
STRIDE = 2000
NSTRIDE = E // STRIDE
CAP = 2080
BCH = 16


def _i16():
    return lax.broadcasted_iota(jnp.int32, (16,), 0)


def _zero_rows_f32(ref, n_rows):
    zv = jnp.zeros((16,), jnp.float32)
    ncols = ref.shape[1] // 16

    def body(i, _):
        ref[i // ncols, pl.ds((i % ncols) * 16, 16)] = zv
        return 0

    lax.fori_loop(0, n_rows * ncols, body, 0)


def _zero_flat_f32(ref, nwords):
    zv = jnp.zeros((16,), jnp.float32)

    def body(i, _):
        ref[pl.ds(i * 16, 16)] = zv
        return 0

    lax.fori_loop(0, nwords // 16, body, 0)


def _compact_stride(dst_st, posc, dstrel, st, lo, hi):
    i16 = _i16()

    def body(i, cnt):
        d16 = dst_st[pl.ds(i * 16, 16)]
        m = (d16 >= lo) & (d16 < hi)
        mi = m.astype(jnp.int32)
        pos = plsc.cumsum(mi)
        tot = jnp.sum(mi)
        tgt = jnp.maximum(cnt + pos - 1, 0)
        gpos = st * STRIDE + i * 16 + i16
        plsc.store_scatter(posc, [tgt], gpos, mask=m)
        plsc.store_scatter(dstrel, [tgt], d16 - lo, mask=m)
        return cnt + tot

    cnt = lax.fori_loop(0, STRIDE // 16, body, jnp.int32(0))
    z = jnp.zeros((16,), jnp.int32)
    pad = jnp.full((16,), RNG, jnp.int32)
    for k in range(BCH // 16):
        idxv = cnt + k * 16 + i16
        plsc.store_scatter(posc, [idxv], z)
        plsc.store_scatter(dstrel, [idxv], pad)
    return (cnt + BCH - 1) // BCH


def _sc_gat_body(xl_hbm, xr_hbm, att_hbm, dst_hbm, src_hbm,
                 num_out, den_out,
                 dst_st, posc, dstrel, srcv, dstab, xlr, xrr, attv,
                 acc, accd, sem1, sem2):
    c = lax.axis_index("c")
    t = lax.axis_index("s")
    w = c * 16 + t
    i16 = _i16()
    pltpu.sync_copy(att_hbm, attv)
    for r in range(NROUND):
        rid = r * NW + w

        def round_body(lo):
            _zero_rows_f32(acc, ACCR)
            _zero_rows_f32(accd, ACCR)

            def sbody(st, _):
                pltpu.sync_copy(dst_hbm.at[pl.ds(st * STRIDE, STRIDE)],
                                dst_st)
                nb = _compact_stride(dst_st, posc, dstrel, st, lo, lo + RNG)

                def bbody(b, _):
                    base = b * BCH
                    rows = i16
                    drel = dstrel[pl.ds(base, 16)]
                    pvec = posc[pl.ds(base, 16)]
                    pltpu.async_copy(src_hbm.at[pvec], srcv, sem1).wait()
                    sv = srcv[...]
                    c1 = pltpu.async_copy(xl_hbm.at[sv], xlr, sem1)
                    c2 = pltpu.async_copy(xr_hbm.at[drel + lo], xrr, sem2)
                    c1.wait()
                    c2.wait()
                    a_hs = []
                    for h in range(HEADS):
                        def dbody(dd, a):
                            for l in range(8):
                                j = h * D_HEAD + dd * 8 + l
                                col = jnp.full((16,), j, jnp.int32)
                                av = plsc.load_gather(attv, [col])
                                u = (plsc.load_gather(xlr, [rows, col])
                                     + plsc.load_gather(xrr, [rows, col]))
                                a = a + jnp.maximum(u, 0.2 * u) * av
                            return a

                        a_hs.append(jnp.exp(lax.fori_loop(
                            0, D_HEAD // 8, dbody,
                            jnp.zeros((16,), jnp.float32))))

                    def ebody(e, _):
                        esel = jnp.full((16,), e, jnp.int32)
                        de = drel[esel]
                        ab = [a_hs[h][esel] for h in range(HEADS)]
                        a_vec = jnp.zeros((16,), jnp.float32)
                        for h in range(HEADS):
                            a_vec = jnp.where(i16 == h, ab[h], a_vec)
                        plsc.addupdate_scatter(accd, [de, i16], a_vec,
                                               mask=i16 < 8)
                        for k in range(D_H // 16):
                            v = xlr[e, pl.ds(k * 16, 16)] * ab[k // 4]
                            plsc.addupdate_scatter(acc, [de, k * 16 + i16], v)
                        return 0

                    lax.fori_loop(0, 16, ebody, 0)
                    return 0

                lax.fori_loop(0, nb, bbody, 0)
                return 0

            lax.fori_loop(0, NSTRIDE, sbody, 0)
            pltpu.sync_copy(acc.at[pl.ds(0, RNG)],
                            num_out.at[pl.ds(lo, RNG)])
            pltpu.sync_copy(accd.at[pl.ds(0, RNG)],
                            den_out.at[pl.ds(lo, RNG)])

        if r < 2:
            round_body(rid * RNG)
        else:
            @pl.when(rid < NRANGE)
            def _():
                round_body(rid * RNG)


def _sc_gat_pass(xl, xr, att_flat, src1, dst1):
    mesh = plsc.VectorSubcoreMesh(core_axis_name="c", subcore_axis_name="s")
    f = functools.partial(
        pl.kernel,
        out_type=[jax.ShapeDtypeStruct((NPAD, D_H), jnp.float32),
                  jax.ShapeDtypeStruct((NPAD, 16), jnp.float32)],
        mesh=mesh,
        scratch_types=[
            pltpu.VMEM((STRIDE,), jnp.int32),
            pltpu.VMEM((CAP,), jnp.int32),
            pltpu.VMEM((CAP,), jnp.int32),
            pltpu.VMEM((16,), jnp.int32),
            pltpu.VMEM((16,), jnp.int32),
            pltpu.VMEM((BCH, D_H), jnp.float32),
            pltpu.VMEM((BCH, D_H), jnp.float32),
            pltpu.VMEM((D_H,), jnp.float32),
            pltpu.VMEM((ACCR, D_H), jnp.float32),
            pltpu.VMEM((ACCR, 16), jnp.float32),
            pltpu.SemaphoreType.DMA,
            pltpu.SemaphoreType.DMA,
        ],
        name="sc_gat_edge_pass",
        compiler_params=pltpu.CompilerParams(needs_layout_passes=False),
    )(_sc_gat_body)
    return f(xl, xr, att_flat, dst1, src1)


def _sc_agg_body(h1_hbm, dst_hbm, src_hbm, agg_out,
                 dst_st, posc, dstrel, srcv, xlr, acc, sem1):
    c = lax.axis_index("c")
    t = lax.axis_index("s")
    w = c * 16 + t
    i16 = _i16()
    for r in range(NROUND):
        rid = r * NW + w

        def round_body(lo):
            _zero_rows_f32(acc, ACCR)

            def sbody(st, _):
                pltpu.sync_copy(dst_hbm.at[pl.ds(st * STRIDE, STRIDE)],
                                dst_st)
                nb = _compact_stride(dst_st, posc, dstrel, st, lo, lo + RNG)

                def bbody(b, _):
                    base = b * BCH
                    rows = i16
                    drel = dstrel[pl.ds(base, 16)]
                    pvec = posc[pl.ds(base, 16)]
                    pltpu.async_copy(src_hbm.at[pvec], srcv, sem1).wait()
                    sv = srcv[...]
                    pltpu.async_copy(h1_hbm.at[sv], xlr, sem1).wait()

                    def ebody(e, _):
                        esel = jnp.full((16,), e, jnp.int32)
                        de = drel[esel]
                        for k in range(D_H // 16):
                            v = xlr[e, pl.ds(k * 16, 16)]
                            plsc.addupdate_scatter(acc, [de, k * 16 + i16], v)
                        return 0

                    lax.fori_loop(0, 16, ebody, 0)
                    return 0

                lax.fori_loop(0, nb, bbody, 0)
                return 0

            lax.fori_loop(0, NSTRIDE, sbody, 0)
            pltpu.sync_copy(acc.at[pl.ds(0, RNG)],
                            agg_out.at[pl.ds(lo, RNG)])

        if r < 2:
            round_body(rid * RNG)
        else:
            @pl.when(rid < NRANGE)
            def _():
                round_body(rid * RNG)


def _sc_agg_pass(h1, src1, dst1):
    mesh = plsc.VectorSubcoreMesh(core_axis_name="c", subcore_axis_name="s")
    f = functools.partial(
        pl.kernel,
        out_type=jax.ShapeDtypeStruct((NPAD, D_H), jnp.float32),
        mesh=mesh,
        scratch_types=[
            pltpu.VMEM((STRIDE,), jnp.int32),
            pltpu.VMEM((CAP,), jnp.int32),
            pltpu.VMEM((CAP,), jnp.int32),
            pltpu.VMEM((16,), jnp.int32),
            pltpu.VMEM((BCH, D_H), jnp.float32),
            pltpu.VMEM((ACCR, D_H), jnp.float32),
            pltpu.SemaphoreType.DMA,
        ],
        name="sc_agg_pass",
        compiler_params=pltpu.CompilerParams(needs_layout_passes=False),
    )(_sc_agg_body)
    return f(h1, dst1, src1)


def _h1_body(num_ref, den_ref, bias_ref, h1_ref):
    parts = []
    for h in range(HEADS):
        dh = den_ref[:, h:h + 1] + 1e-16
        parts.append(num_ref[:, h * D_HEAD:(h + 1) * D_HEAD] / dh)
    h1_ref[...] = jnp.concatenate(parts, axis=-1) + bias_ref[...]


def _h1_build(num, den, gat_bias):
    return pl.pallas_call(
        _h1_body,
        grid=(N_BLOCKS,),
        in_specs=[
            pl.BlockSpec((NB, D_H), lambda i: (i, 0)),
            pl.BlockSpec((NB, 16), lambda i: (i, 0)),
            pl.BlockSpec((D_H,), lambda i: (0,)),
        ],
        out_specs=pl.BlockSpec((NB, D_H), lambda i: (i, 0)),
        out_shape=jax.ShapeDtypeStruct((N, D_H), jnp.float32),
    )(num, den, gat_bias)


def _edge_phase(xl, xr, gat_att, gat_bias, edge_index):
    src = edge_index[0]
    dst = edge_index[1]
    xl3 = xl.reshape(N, HEADS, D_HEAD)
    xr3 = xr.reshape(N, HEADS, D_HEAD)
    e = _leaky(xl3[src] + xr3[dst])
    logits = (e * gat_att[None, :, :]).sum(-1)
    a = jnp.exp(logits)
    denom = jax.ops.segment_sum(a, dst, num_segments=N)
    num = jax.ops.segment_sum(xl3[src] * a[:, :, None], dst, num_segments=N)
    h1 = (num / (denom[:, :, None] + 1e-16)).reshape(N, D_H) + gat_bias
    agg = jax.ops.segment_sum(h1[src], dst, num_segments=N)
    return h1, agg


def kernel(x, fp_w, fp_b, fp_bn_g, fp_bn_b, gat_wl, gat_wr, gat_att, gat_bias,
           gin_w, gin_b, gin_bn_g, gin_bn_b, ln_g, ln_b, res_w, res_b,
           gate_w1, gate_b1, gate_w2, gate_b2, h_qkv_w, h_qkv_b, h_out_w,
           h_out_b, h_w1, h_b1, h_bn_g, h_bn_b, h_w2, h_b2, edge_index, batch):
    xl, xr = _featurize(x, fp_w, fp_b, fp_bn_g, fp_bn_b, gat_wl, gat_wr)
    src1 = edge_index[0]
    dst1 = edge_index[1]
    num, den = _sc_gat_pass(xl, xr, gat_att.reshape(D_H), src1, dst1)
    h1 = _h1_build(num[:N], den[:N], gat_bias)
    agg = _sc_agg_pass(h1, src1, dst1)[:N]
    batch3 = batch.reshape(N_BLOCKS, 1, NB)
    embnum, gs = _node_mlp_pool(h1, agg, batch3, gin_w, gin_b, gin_bn_g,
                                gin_bn_b, ln_g, ln_b, res_w, res_b,
                                gate_w1, gate_b1, gate_w2, gate_b2)
    return _label_heads(embnum, gs, h_qkv_w, h_qkv_b, h_out_w, h_out_b,
                        h_w1, h_b1, h_bn_g, h_bn_b, h_w2, h_b2)
